# baseline (speedup 1.0000x reference)
import functools

import jax
import jax.numpy as jnp
from jax import lax
from jax.experimental import pallas as pl
from jax.experimental.pallas import tpu as pltpu
from jax.experimental.pallas import tpu_sc as plsc

N_NODES = 10000
N_EDGES = 320000
D = 128

NC = 2
NS = 16
NW = NC * NS
B = 64
MCH = 158
E_PAD = NW * MCH * B
N_PAD = 10240
RPS = N_PAD // NS
ZCH = 64
CW = 8


def _sc_agg_body(with_count, *refs):
    if with_count:
        (x_hbm, src_hbm, dst_hbm, z128_hbm, z8_hbm, ones_hbm,
         part_hbm, cntp_hbm,
         agg_sh, cnt_sh, idxs_v, idxd_v, rows_v, ones_v, sem0, sem1) = refs
    else:
        (x_hbm, src_hbm, dst_hbm, z128_hbm,
         part_hbm,
         agg_sh, idxs_v, idxd_v, rows_v, sem0, sem1) = refs

    c = lax.axis_index("c")
    s = lax.axis_index("s")
    wid = s * NC + c

    pltpu.sync_copy(z128_hbm, rows_v.at[0])
    for k in range(RPS // ZCH):
        pltpu.sync_copy(rows_v.at[0], agg_sh.at[pl.ds(s * RPS + k * ZCH, ZCH)])
    if with_count:
        pltpu.sync_copy(z8_hbm, ones_v)
        for k in range(RPS // ZCH):
            pltpu.sync_copy(ones_v, cnt_sh.at[pl.ds(s * RPS + k * ZCH, ZCH)])
        pltpu.sync_copy(ones_hbm, ones_v)

    pltpu.sync_copy(src_hbm.at[wid], idxs_v)
    pltpu.sync_copy(dst_hbm.at[wid], idxd_v)

    plsc.subcore_barrier()

    pltpu.async_copy(x_hbm.at[idxs_v.at[0]], rows_v.at[0], sem0)

    def pair(g, carry):
        for bi, (sw, so) in enumerate(((sem0, sem1), (sem1, sem0))):
            j = 2 * g + bi
            pltpu.make_async_copy(
                x_hbm.at[idxs_v.at[j]], rows_v.at[bi], sw).wait()
            pltpu.async_copy(
                x_hbm.at[idxs_v.at[j + 1]], rows_v.at[1 - bi], so)
            pltpu.sync_copy(rows_v.at[bi], agg_sh.at[idxd_v.at[j]], add=True)
            if with_count:
                pltpu.sync_copy(ones_v, cnt_sh.at[idxd_v.at[j]], add=True)
        return carry

    lax.fori_loop(0, MCH // 2, pair, 0)
    pltpu.make_async_copy(
        x_hbm.at[idxs_v.at[MCH]], rows_v.at[0], sem0).wait()

    plsc.subcore_barrier()

    for k in range(RPS // ZCH):
        r0 = s * RPS + k * ZCH
        pltpu.sync_copy(agg_sh.at[pl.ds(r0, ZCH)], rows_v.at[0])
        pltpu.sync_copy(rows_v.at[0], part_hbm.at[c, pl.ds(r0, ZCH)])
        if with_count:
            pltpu.sync_copy(cnt_sh.at[pl.ds(r0, ZCH)], ones_v)
            pltpu.sync_copy(ones_v, cntp_hbm.at[c, pl.ds(r0, ZCH)])


def _make_sc_agg(with_count):
    mesh = plsc.VectorSubcoreMesh(
        core_axis_name="c", subcore_axis_name="s",
        num_cores=NC, num_subcores=NS)
    if with_count:
        out_type = (
            jax.ShapeDtypeStruct((NC, N_PAD, D), jnp.float32),
            jax.ShapeDtypeStruct((NC, N_PAD, CW), jnp.float32),
        )
        scratch = [
            pltpu.VMEM_SHARED((N_PAD, D), jnp.float32),
            pltpu.VMEM_SHARED((N_PAD, CW), jnp.float32),
            pltpu.VMEM((MCH + 1, B), jnp.int32),
            pltpu.VMEM((MCH + 1, B), jnp.int32),
            pltpu.VMEM((2, B, D), jnp.float32),
            pltpu.VMEM((B, CW), jnp.float32),
            pltpu.SemaphoreType.DMA,
            pltpu.SemaphoreType.DMA,
        ]
    else:
        out_type = jax.ShapeDtypeStruct((NC, N_PAD, D), jnp.float32)
        scratch = [
            pltpu.VMEM_SHARED((N_PAD, D), jnp.float32),
            pltpu.VMEM((MCH + 1, B), jnp.int32),
            pltpu.VMEM((MCH + 1, B), jnp.int32),
            pltpu.VMEM((2, B, D), jnp.float32),
            pltpu.SemaphoreType.DMA,
            pltpu.SemaphoreType.DMA,
        ]
    return pl.kernel(
        functools.partial(_sc_agg_body, with_count),
        out_type=out_type, mesh=mesh, scratch_types=scratch,
        compiler_params=pltpu.CompilerParams(use_tc_tiling_on_sc=False),
        name=f"sage_sc_agg_cnt{int(with_count)}")


_R = 1000


def _tc_dense_body(p0, p1, c0, c1, x, wl, wr, b, o):
    agg = p0[...] + p1[...]
    cnt = c0[:, 0:1] + c1[:, 0:1]
    inv = 1.0 / jnp.maximum(cnt, 1.0)
    g = jnp.dot(agg, wl[...], preferred_element_type=jnp.float32)
    h = jnp.dot(x[...], wr[...], preferred_element_type=jnp.float32)
    o[...] = jnp.maximum(g * inv + h + b[...], 0.0)


def _tc_dense(part, cntp, x, wl, wr, b):
    grid = (N_NODES // _R,)
    row = pl.BlockSpec((_R, D), lambda i: (i, 0))
    cb = pl.BlockSpec((_R, CW), lambda i: (i, 0))
    full = pl.BlockSpec((D, D), lambda i: (0, 0))
    bias = pl.BlockSpec((1, D), lambda i: (0, 0))
    return pl.pallas_call(
        _tc_dense_body,
        grid=grid,
        in_specs=[row, row, cb, cb, row, full, full, bias],
        out_specs=row,
        out_shape=jax.ShapeDtypeStruct((N_NODES, D), jnp.float32),
    )(part[0], part[1], cntp[0], cntp[1], x, wl, wr, b.reshape(1, D))


def kernel(x, edge_index, Wl1, Wr1, b1, Wl2, Wr2, b2):
    n_extra = E_PAD - N_EDGES
    src = edge_index[0].astype(jnp.int32)
    dst = edge_index[1].astype(jnp.int32)
    src = jnp.concatenate([src, jnp.zeros((n_extra,), jnp.int32)])
    pad_dst = N_NODES + (jnp.arange(n_extra, dtype=jnp.int32) % (N_PAD - N_NODES))
    dst = jnp.concatenate([dst, pad_dst])
    src = src.reshape(NW, MCH, B)
    dst = dst.reshape(NW, MCH, B)
    dummy = jnp.zeros((NW, 1, B), jnp.int32)
    src = jnp.concatenate([src, dummy], axis=1)
    dst = jnp.concatenate([dst, dummy], axis=1)
    x = x.astype(jnp.float32)
    z128 = jnp.zeros((ZCH, D), jnp.float32)
    z8 = jnp.zeros((ZCH, CW), jnp.float32)
    ones = jnp.ones((B, CW), jnp.float32)

    part1, cntp = _make_sc_agg(True)(x, src, dst, z128, z8, ones)
    h = _tc_dense(part1, cntp, x, Wl1, Wr1, b1)
    part2 = _make_sc_agg(False)(h, src, dst, z128)
    out = _tc_dense(part2, cntp, h, Wl2, Wr2, b2)
    return out


# speedup vs baseline: 1.1128x; 1.1128x over previous
import functools

import jax
import jax.numpy as jnp
from jax import lax
from jax.experimental import pallas as pl
from jax.experimental.pallas import tpu as pltpu
from jax.experimental.pallas import tpu_sc as plsc

N_NODES = 10000
N_EDGES = 320000
D = 128

NC = 2
NS = 16
B = 128
MCHT = 158
MC0 = 57
MC1 = MCHT - MC0
E_PAD = NS * MCHT * B
N_PAD = 10240
RPS = N_PAD // NS
ZCH = 128
CW = 8


def _sc_agg_body(with_count, *refs):
    if with_count:
        (x_hbm, src_hbm, dst_hbm, z128_hbm, z8_hbm, ones_hbm,
         part_hbm, cntp_hbm,
         agg_sh, cnt_sh, idxs_v, idxd_v, rows_v, ones_v, sem) = refs
    else:
        (x_hbm, src_hbm, dst_hbm, z128_hbm,
         part_hbm,
         agg_sh, idxs_v, idxd_v, rows_v, sem) = refs

    c = lax.axis_index("c")
    s = lax.axis_index("s")

    pltpu.sync_copy(z128_hbm, rows_v)
    for k in range(RPS // ZCH):
        pltpu.sync_copy(rows_v, agg_sh.at[pl.ds(s * RPS + k * ZCH, ZCH)])
    if with_count:
        pltpu.sync_copy(z8_hbm, ones_v)
        for k in range(RPS // ZCH):
            pltpu.sync_copy(ones_v, cnt_sh.at[pl.ds(s * RPS + k * ZCH, ZCH)])
        pltpu.sync_copy(ones_hbm, ones_v)

    @pl.when(c == 0)
    def _():
        pltpu.sync_copy(src_hbm.at[s, pl.ds(0, MC0)], idxs_v.at[pl.ds(0, MC0)])
        pltpu.sync_copy(dst_hbm.at[s, pl.ds(0, MC0)], idxd_v.at[pl.ds(0, MC0)])

    @pl.when(c == 1)
    def _():
        pltpu.sync_copy(src_hbm.at[s, pl.ds(MC0, MC1)], idxs_v)
        pltpu.sync_copy(dst_hbm.at[s, pl.ds(MC0, MC1)], idxd_v)

    nch = lax.select(c == 0, MC0, MC1)

    plsc.subcore_barrier()

    def chunk(j, carry):
        pltpu.async_copy(x_hbm.at[idxs_v.at[j]], rows_v, sem).wait()
        pltpu.sync_copy(rows_v, agg_sh.at[idxd_v.at[j]], add=True)
        if with_count:
            pltpu.sync_copy(ones_v, cnt_sh.at[idxd_v.at[j]], add=True)
        return carry

    lax.fori_loop(0, nch, chunk, 0)

    plsc.subcore_barrier()

    for k in range(RPS // ZCH):
        r0 = s * RPS + k * ZCH
        pltpu.sync_copy(agg_sh.at[pl.ds(r0, ZCH)], rows_v)
        pltpu.sync_copy(rows_v, part_hbm.at[c, pl.ds(r0, ZCH)])
        if with_count:
            pltpu.sync_copy(cnt_sh.at[pl.ds(r0, ZCH)], ones_v)
            pltpu.sync_copy(ones_v, cntp_hbm.at[c, pl.ds(r0, ZCH)])


def _make_sc_agg(with_count):
    mesh = plsc.VectorSubcoreMesh(
        core_axis_name="c", subcore_axis_name="s",
        num_cores=NC, num_subcores=NS)
    if with_count:
        out_type = (
            jax.ShapeDtypeStruct((NC, N_PAD, D), jnp.float32),
            jax.ShapeDtypeStruct((NC, N_PAD, CW), jnp.float32),
        )
        scratch = [
            pltpu.VMEM_SHARED((N_PAD, D), jnp.float32),
            pltpu.VMEM_SHARED((N_PAD, CW), jnp.float32),
            pltpu.VMEM((MC1, B), jnp.int32),
            pltpu.VMEM((MC1, B), jnp.int32),
            pltpu.VMEM((ZCH, D), jnp.float32),
            pltpu.VMEM((ZCH, CW), jnp.float32),
            pltpu.SemaphoreType.DMA,
        ]
    else:
        out_type = jax.ShapeDtypeStruct((NC, N_PAD, D), jnp.float32)
        scratch = [
            pltpu.VMEM_SHARED((N_PAD, D), jnp.float32),
            pltpu.VMEM((MC1, B), jnp.int32),
            pltpu.VMEM((MC1, B), jnp.int32),
            pltpu.VMEM((ZCH, D), jnp.float32),
            pltpu.SemaphoreType.DMA,
        ]
    return pl.kernel(
        functools.partial(_sc_agg_body, with_count),
        out_type=out_type, mesh=mesh, scratch_types=scratch,
        compiler_params=pltpu.CompilerParams(use_tc_tiling_on_sc=False),
        name=f"sage_sc_agg_cnt{int(with_count)}")


_R = 1000


def _tc_dense_body(p0, p1, c0, c1, x, wl, wr, b, o):
    agg = p0[...] + p1[...]
    cnt = c0[:, 0:1] + c1[:, 0:1]
    inv = 1.0 / jnp.maximum(cnt, 1.0)
    g = jnp.dot(agg, wl[...], preferred_element_type=jnp.float32)
    h = jnp.dot(x[...], wr[...], preferred_element_type=jnp.float32)
    o[...] = jnp.maximum(g * inv + h + b[...], 0.0)


def _tc_dense(part, cntp, x, wl, wr, b):
    grid = (N_NODES // _R,)
    row = pl.BlockSpec((_R, D), lambda i: (i, 0))
    cb = pl.BlockSpec((_R, CW), lambda i: (i, 0))
    full = pl.BlockSpec((D, D), lambda i: (0, 0))
    bias = pl.BlockSpec((1, D), lambda i: (0, 0))
    return pl.pallas_call(
        _tc_dense_body,
        grid=grid,
        in_specs=[row, row, cb, cb, row, full, full, bias],
        out_specs=row,
        out_shape=jax.ShapeDtypeStruct((N_NODES, D), jnp.float32),
    )(part[0], part[1], cntp[0], cntp[1], x, wl, wr, b.reshape(1, D))


def kernel(x, edge_index, Wl1, Wr1, b1, Wl2, Wr2, b2):
    n_extra = E_PAD - N_EDGES
    src = edge_index[0].astype(jnp.int32)
    dst = edge_index[1].astype(jnp.int32)
    src = jnp.concatenate([src, jnp.zeros((n_extra,), jnp.int32)])
    pad_dst = N_NODES + (jnp.arange(n_extra, dtype=jnp.int32) % (N_PAD - N_NODES))
    dst = jnp.concatenate([dst, pad_dst])
    src = src.reshape(NS, MCHT, B)
    dst = dst.reshape(NS, MCHT, B)
    x = x.astype(jnp.float32)
    z128 = jnp.zeros((ZCH, D), jnp.float32)
    z8 = jnp.zeros((ZCH, CW), jnp.float32)
    ones = jnp.ones((B, CW), jnp.float32)

    part1, cntp = _make_sc_agg(True)(x, src, dst, z128, z8, ones)
    h = _tc_dense(part1, cntp, x, Wl1, Wr1, b1)
    part2 = _make_sc_agg(False)(h, src, dst, z128)
    out = _tc_dense(part2, cntp, h, Wl2, Wr2, b2)
    return out


# speedup vs baseline: 1.3811x; 1.2411x over previous
import functools

import jax
import jax.numpy as jnp
from jax import lax
from jax.experimental import pallas as pl
from jax.experimental.pallas import tpu as pltpu
from jax.experimental.pallas import tpu_sc as plsc

N_NODES = 10000
N_EDGES = 320000
D = 128

NC = 2
NS = 16
B = 128
MCHT = 158
MC0 = 101
MC1 = MCHT - MC0
MCX = max(MC0, MC1)
E_PAD = NS * MCHT * B
N_PAD = 10240
RPS = N_PAD // NS
ZCH = 128
CW = 8


def _sc_agg_body(with_count, *refs):
    if with_count:
        (x_hbm, src_hbm, dst_hbm, z128_hbm, z8_hbm, ones_hbm,
         part_hbm, cntp_hbm,
         agg_sh, cnt_sh, idxs_v, idxd_v, rows_v, ones_v, sem) = refs
    else:
        (x_hbm, src_hbm, dst_hbm, z128_hbm,
         part_hbm,
         agg_sh, idxs_v, idxd_v, rows_v, sem) = refs

    c = lax.axis_index("c")
    s = lax.axis_index("s")

    pltpu.sync_copy(z128_hbm, rows_v)
    for k in range(RPS // ZCH):
        pltpu.sync_copy(rows_v, agg_sh.at[pl.ds(s * RPS + k * ZCH, ZCH)])
    if with_count:
        pltpu.sync_copy(z8_hbm, ones_v)
        for k in range(RPS // ZCH):
            pltpu.sync_copy(ones_v, cnt_sh.at[pl.ds(s * RPS + k * ZCH, ZCH)])
        pltpu.sync_copy(ones_hbm, ones_v)

    @pl.when(c == 0)
    def _():
        pltpu.sync_copy(src_hbm.at[s, pl.ds(0, MC0)], idxs_v.at[pl.ds(0, MC0)])
        pltpu.sync_copy(dst_hbm.at[s, pl.ds(0, MC0)], idxd_v.at[pl.ds(0, MC0)])

    @pl.when(c == 1)
    def _():
        pltpu.sync_copy(src_hbm.at[s, pl.ds(MC0, MC1)], idxs_v.at[pl.ds(0, MC1)])
        pltpu.sync_copy(dst_hbm.at[s, pl.ds(MC0, MC1)], idxd_v.at[pl.ds(0, MC1)])

    nch = lax.select(c == 0, MC0, MC1)

    plsc.subcore_barrier()

    def chunk(j, carry):
        pltpu.async_copy(x_hbm.at[idxs_v.at[j]], rows_v, sem).wait()
        pltpu.sync_copy(rows_v, agg_sh.at[idxd_v.at[j]], add=True)
        if with_count:
            pltpu.sync_copy(ones_v, cnt_sh.at[idxd_v.at[j]], add=True)
        return carry

    lax.fori_loop(0, nch, chunk, 0)

    plsc.subcore_barrier()

    for k in range(RPS // ZCH):
        r0 = s * RPS + k * ZCH
        pltpu.sync_copy(agg_sh.at[pl.ds(r0, ZCH)], rows_v)
        pltpu.sync_copy(rows_v, part_hbm.at[c, pl.ds(r0, ZCH)])
        if with_count:
            pltpu.sync_copy(cnt_sh.at[pl.ds(r0, ZCH)], ones_v)
            pltpu.sync_copy(ones_v, cntp_hbm.at[c, pl.ds(r0, ZCH)])


def _make_sc_agg(with_count):
    mesh = plsc.VectorSubcoreMesh(
        core_axis_name="c", subcore_axis_name="s",
        num_cores=NC, num_subcores=NS)
    if with_count:
        out_type = (
            jax.ShapeDtypeStruct((NC, N_PAD, D), jnp.float32),
            jax.ShapeDtypeStruct((NC, N_PAD, CW), jnp.float32),
        )
        scratch = [
            pltpu.VMEM_SHARED((N_PAD, D), jnp.float32),
            pltpu.VMEM_SHARED((N_PAD, CW), jnp.float32),
            pltpu.VMEM((MCX, B), jnp.int32),
            pltpu.VMEM((MCX, B), jnp.int32),
            pltpu.VMEM((ZCH, D), jnp.float32),
            pltpu.VMEM((ZCH, CW), jnp.float32),
            pltpu.SemaphoreType.DMA,
        ]
    else:
        out_type = jax.ShapeDtypeStruct((NC, N_PAD, D), jnp.float32)
        scratch = [
            pltpu.VMEM_SHARED((N_PAD, D), jnp.float32),
            pltpu.VMEM((MCX, B), jnp.int32),
            pltpu.VMEM((MCX, B), jnp.int32),
            pltpu.VMEM((ZCH, D), jnp.float32),
            pltpu.SemaphoreType.DMA,
        ]
    return pl.kernel(
        functools.partial(_sc_agg_body, with_count),
        out_type=out_type, mesh=mesh, scratch_types=scratch,
        compiler_params=pltpu.CompilerParams(use_tc_tiling_on_sc=False),
        name=f"sage_sc_agg_cnt{int(with_count)}")


_R = 1000


def _tc_dense_body(p0, p1, c0, c1, x, wl, wr, b, o):
    agg = p0[...] + p1[...]
    cnt = c0[:, 0:1] + c1[:, 0:1]
    inv = 1.0 / jnp.maximum(cnt, 1.0)
    g = jnp.dot(agg, wl[...], preferred_element_type=jnp.float32)
    h = jnp.dot(x[...], wr[...], preferred_element_type=jnp.float32)
    o[...] = jnp.maximum(g * inv + h + b[...], 0.0)


def _tc_dense(part, cntp, x, wl, wr, b):
    grid = (N_NODES // _R,)
    row = pl.BlockSpec((_R, D), lambda i: (i, 0))
    cb = pl.BlockSpec((_R, CW), lambda i: (i, 0))
    full = pl.BlockSpec((D, D), lambda i: (0, 0))
    bias = pl.BlockSpec((1, D), lambda i: (0, 0))
    return pl.pallas_call(
        _tc_dense_body,
        grid=grid,
        in_specs=[row, row, cb, cb, row, full, full, bias],
        out_specs=row,
        out_shape=jax.ShapeDtypeStruct((N_NODES, D), jnp.float32),
    )(part[0], part[1], cntp[0], cntp[1], x, wl, wr, b.reshape(1, D))


def kernel(x, edge_index, Wl1, Wr1, b1, Wl2, Wr2, b2):
    n_extra = E_PAD - N_EDGES
    src = edge_index[0].astype(jnp.int32)
    dst = edge_index[1].astype(jnp.int32)
    src = jnp.concatenate([src, jnp.zeros((n_extra,), jnp.int32)])
    pad_dst = N_NODES + (jnp.arange(n_extra, dtype=jnp.int32) % (N_PAD - N_NODES))
    dst = jnp.concatenate([dst, pad_dst])
    src = src.reshape(NS, MCHT, B)
    dst = dst.reshape(NS, MCHT, B)
    x = x.astype(jnp.float32)
    z128 = jnp.zeros((ZCH, D), jnp.float32)
    z8 = jnp.zeros((ZCH, CW), jnp.float32)
    ones = jnp.ones((B, CW), jnp.float32)

    part1, cntp = _make_sc_agg(True)(x, src, dst, z128, z8, ones)
    h = _tc_dense(part1, cntp, x, Wl1, Wr1, b1)
    part2 = _make_sc_agg(False)(h, src, dst, z128)
    out = _tc_dense(part2, cntp, h, Wl2, Wr2, b2)
    return out
